# Initial kernel scaffold; baseline (speedup 1.0000x reference)
import functools
import jax
import jax.numpy as jnp
from jax import lax
from jax.experimental import pallas as pl
from jax.experimental.pallas import tpu as pltpu
from jax.experimental.pallas import tpu_sc as plsc

D = 64
B = 4096 * 200
NC, NS = 2, 16
NW = NC * NS
BPW = B // NW
CHUNK = 512
NCHUNK = BPW // CHUNK

_mesh = plsc.VectorSubcoreMesh(core_axis_name="c", subcore_axis_name="s")


@functools.partial(
    pl.kernel,
    mesh=_mesh,
    out_type=jax.ShapeDtypeStruct((B, D), jnp.float32),
    scratch_types=[
        pltpu.VMEM((BPW,), jnp.int32),
        pltpu.VMEM((2, CHUNK, D), jnp.float32),
        pltpu.SemaphoreType.DMA,
        pltpu.SemaphoreType.DMA,
    ],
)
def _gather_kernel(table_hbm, idx_hbm, out_hbm, idx_v, rows_v, gsem, wsem):
    wid = lax.axis_index("s") * NC + lax.axis_index("c")
    base = wid * BPW
    pltpu.sync_copy(idx_hbm.at[pl.ds(base, BPW)], idx_v)

    def chunk(i, carry):
        for b in range(2):
            off = (2 * i + b) * CHUNK
            pltpu.async_copy(
                table_hbm.at[idx_v.at[pl.ds(off, CHUNK)]], rows_v.at[b], gsem
            ).wait()
            pltpu.async_copy(
                rows_v.at[b], out_hbm.at[pl.ds(base + off, CHUNK)], wsem
            ).wait()
        return carry

    lax.fori_loop(0, NCHUNK // 2, chunk, 0)


def kernel(joint_indices, table):
    n, m = joint_indices.shape
    flat_idx = joint_indices.reshape(n * m).astype(jnp.int32)
    out = _gather_kernel(table, flat_idx)
    return out.reshape(n, m, D)


# speedup vs baseline: 4.0867x; 4.0867x over previous
import functools
import jax
import jax.numpy as jnp
from jax import lax
from jax.experimental import pallas as pl
from jax.experimental.pallas import tpu as pltpu
from jax.experimental.pallas import tpu_sc as plsc

D = 64
B = 4096 * 200
NC, NS = 2, 16
NW = NC * NS
BPW = B // NW
CHUNK = 512
NCHUNK = BPW // CHUNK

_mesh = plsc.VectorSubcoreMesh(core_axis_name="c", subcore_axis_name="s")


@functools.partial(
    pl.kernel,
    mesh=_mesh,
    out_type=jax.ShapeDtypeStruct((B, D), jnp.float32),
    scratch_types=[
        pltpu.VMEM((BPW,), jnp.int32),
        pltpu.VMEM((2, CHUNK, D), jnp.float32),
        pltpu.SemaphoreType.DMA,
        pltpu.SemaphoreType.DMA,
    ],
    compiler_params=pltpu.CompilerParams(use_tc_tiling_on_sc=False),
)
def _gather_kernel(table_hbm, idx_hbm, out_hbm, idx_v, rows_v, gsem, wsem):
    wid = lax.axis_index("s") * NC + lax.axis_index("c")
    base = wid * BPW
    pltpu.sync_copy(idx_hbm.at[pl.ds(base, BPW)], idx_v)

    def chunk(i, carry):
        for b in range(2):
            off = (2 * i + b) * CHUNK
            pltpu.async_copy(
                table_hbm.at[idx_v.at[pl.ds(off, CHUNK)]], rows_v.at[b], gsem
            ).wait()
            pltpu.async_copy(
                rows_v.at[b], out_hbm.at[pl.ds(base + off, CHUNK)], wsem
            ).wait()
        return carry

    lax.fori_loop(0, NCHUNK // 2, chunk, 0)


def kernel(joint_indices, table):
    n, m = joint_indices.shape
    flat_idx = joint_indices.reshape(n * m).astype(jnp.int32)
    out = _gather_kernel(table, flat_idx)
    return out.reshape(n, m, D)


# speedup vs baseline: 4.2594x; 1.0423x over previous
import functools
import jax
import jax.numpy as jnp
from jax import lax
from jax.experimental import pallas as pl
from jax.experimental.pallas import tpu as pltpu
from jax.experimental.pallas import tpu_sc as plsc

D = 64
B = 4096 * 200
NC, NS = 2, 16
NW = NC * NS
BPW = B // NW
CHUNK = 256
N = BPW // CHUNK
NBUF = 4

_mesh = plsc.VectorSubcoreMesh(core_axis_name="c", subcore_axis_name="s")


@functools.partial(
    pl.kernel,
    mesh=_mesh,
    out_type=jax.ShapeDtypeStruct((B, D), jnp.float32),
    scratch_types=[
        pltpu.VMEM((BPW,), jnp.int32),
        pltpu.VMEM((NBUF, CHUNK, D), jnp.float32),
        pltpu.SemaphoreType.DMA((NBUF,)),
        pltpu.SemaphoreType.DMA((NBUF,)),
    ],
    compiler_params=pltpu.CompilerParams(use_tc_tiling_on_sc=False),
)
def _gather_kernel(table_hbm, idx_hbm, out_hbm, idx_v, rows_v, gsem, wsem):
    wid = lax.axis_index("s") * NC + lax.axis_index("c")
    base = wid * BPW
    pltpu.sync_copy(idx_hbm.at[pl.ds(base, BPW)], idx_v)

    def gather(g, j):
        return pltpu.make_async_copy(
            table_hbm.at[idx_v.at[pl.ds(g * CHUNK, CHUNK)]],
            rows_v.at[j],
            gsem.at[j],
        )

    def writeback(g, j):
        return pltpu.make_async_copy(
            rows_v.at[j],
            out_hbm.at[pl.ds(base + g * CHUNK, CHUNK)],
            wsem.at[j],
        )

    def step(g, j, wait_wb, prefetch):
        gather(g, j).wait()
        writeback(g, j).start()
        if wait_wb:
            writeback(g - 2, (j + 2) % NBUF).wait()
        if prefetch:
            gather(g + 2, (j + 2) % NBUF).start()

    gather(0, 0).start()
    gather(1, 1).start()

    step(0, 0, False, True)
    step(1, 1, False, True)
    step(2, 2, True, True)
    step(3, 3, True, True)

    def block(i, carry):
        g0 = i * NBUF
        for j in range(NBUF):
            step(g0 + j, j, True, True)
        return carry

    lax.fori_loop(1, N // NBUF - 1, block, 0)

    step(N - 4, 0, True, True)
    step(N - 3, 1, True, True)
    step(N - 2, 2, True, False)
    step(N - 1, 3, True, False)

    writeback(N - 2, 2).wait()
    writeback(N - 1, 3).wait()


def kernel(joint_indices, table):
    n, m = joint_indices.shape
    flat_idx = joint_indices.reshape(n * m).astype(jnp.int32)
    out = _gather_kernel(table, flat_idx)
    return out.reshape(n, m, D)


# speedup vs baseline: 4.2627x; 1.0008x over previous
import functools
import jax
import jax.numpy as jnp
from jax import lax
from jax.experimental import pallas as pl
from jax.experimental.pallas import tpu as pltpu
from jax.experimental.pallas import tpu_sc as plsc

NI, NJ = 4096, 200
D = 64
B = NI * NJ
NC, NS = 2, 16
NW = NC * NS
BPW = B // NW
IPW = NI // NW
CHUNK = NJ
N = IPW
NBUF = 4

_mesh = plsc.VectorSubcoreMesh(core_axis_name="c", subcore_axis_name="s")


@functools.partial(
    pl.kernel,
    mesh=_mesh,
    out_type=jax.ShapeDtypeStruct((NI, NJ, D), jnp.float32),
    scratch_types=[
        pltpu.VMEM((BPW,), jnp.int32),
        pltpu.VMEM((NBUF, CHUNK, D), jnp.float32),
        pltpu.SemaphoreType.DMA((NBUF,)),
        pltpu.SemaphoreType.DMA((NBUF,)),
    ],
    compiler_params=pltpu.CompilerParams(use_tc_tiling_on_sc=False),
)
def _gather_kernel(table_hbm, idx_hbm, out_hbm, idx_v, rows_v, gsem, wsem):
    wid = lax.axis_index("s") * NC + lax.axis_index("c")
    base = wid * BPW
    i0 = wid * IPW
    pltpu.sync_copy(idx_hbm.at[pl.ds(base, BPW)], idx_v)

    def gather(g, j):
        return pltpu.make_async_copy(
            table_hbm.at[idx_v.at[pl.ds(g * CHUNK, CHUNK)]],
            rows_v.at[j],
            gsem.at[j],
        )

    def writeback(g, j):
        return pltpu.make_async_copy(
            rows_v.at[j],
            out_hbm.at[i0 + g],
            wsem.at[j],
        )

    def step(g, j, wait_wb, prefetch):
        gather(g, j).wait()
        writeback(g, j).start()
        if wait_wb:
            writeback(g - 2, (j + 2) % NBUF).wait()
        if prefetch:
            gather(g + 2, (j + 2) % NBUF).start()

    gather(0, 0).start()
    gather(1, 1).start()

    step(0, 0, False, True)
    step(1, 1, False, True)
    step(2, 2, True, True)
    step(3, 3, True, True)

    def block(i, carry):
        g0 = i * NBUF
        for j in range(NBUF):
            step(g0 + j, j, True, True)
        return carry

    lax.fori_loop(1, N // NBUF - 1, block, 0)

    step(N - 4, 0, True, True)
    step(N - 3, 1, True, True)
    step(N - 2, 2, True, False)
    step(N - 1, 3, True, False)

    writeback(N - 2, 2).wait()
    writeback(N - 1, 3).wait()


def kernel(joint_indices, table):
    flat_idx = joint_indices.reshape(B).astype(jnp.int32)
    return _gather_kernel(table, flat_idx)
